# baseline (speedup 1.0000x reference)
import numpy as np
import jax
import jax.numpy as jnp
from jax.experimental import pallas as pl
from jax.experimental.pallas import tpu as pltpu

_GAMMA = 0.9865
_DELTA = _GAMMA * _GAMMA
_CHUNK = 256
_SEQ = 1024
_BPG = 2


def _retention_body(xq_ref, xk_ref, xv_ref, wq_ref, wk_ref, wv_ref,
                    dmat_ref, qdec_ref, kdec_ref, out_ref,
                    wqa_s, wka_s, wv_s):
    C = _CHUNK
    f32 = jnp.float32
    bf16 = jnp.bfloat16
    D = wq_ref.shape[0]

    @pl.when(pl.program_id(0) == 0)
    def _():
        ones = jnp.ones((D, 128), bf16)
        wq = wq_ref[...].astype(bf16)
        wk = wk_ref[...].astype(bf16)
        wqa_s[:, :D] = wq
        wqa_s[:, D:] = jnp.dot(wq, ones, preferred_element_type=f32
                               ).astype(bf16)
        wka_s[:, :D] = wk
        wka_s[:, D:] = jnp.dot(wk, ones, preferred_element_type=f32
                               ).astype(bf16)
        wv_s[...] = wv_ref[...].astype(bf16)

    qf = jnp.dot(xq_ref[0].astype(bf16), wqa_s[...],
                 preferred_element_type=f32)
    kf = jnp.dot(xk_ref[0].astype(bf16), wka_s[...],
                 preferred_element_type=f32)
    vf = jnp.dot(xv_ref[0].astype(bf16), wv_s[...],
                 preferred_element_type=f32)
    q = qf[:, :D].astype(bf16)
    k = kf[:, :D].astype(bf16)
    v = vf.astype(bf16)
    qk = qf[:, D:] * kf[:, D:]
    S = q.shape[0]
    N = S // C

    qs_all = jnp.concatenate([q[1:], jnp.zeros((1, D), bf16)], axis=0)

    rows = jax.lax.broadcasted_iota(jnp.int32, (C, C), 0)
    cols = jax.lax.broadcasted_iota(jnp.int32, (C, C), 1)
    eye = rows == cols
    dmat = dmat_ref[...]
    qdec = qdec_ref[...].astype(bf16)
    kdec = kdec_ref[...].astype(bf16)
    dc = jnp.array(_DELTA ** C, bf16)
    half = _SEQ // C

    r = jnp.zeros((D, D), bf16)
    for c in range(N):
        if c % half == 0:
            r = jnp.zeros((D, D), bf16)
        sl = slice(c * C, (c + 1) * C)
        qs = qs_all[sl]
        kc = k[sl]
        vc = v[sl]

        a = jax.lax.dot_general(qs, kc, (((1,), (1,)), ((), ())),
                                preferred_element_type=f32)
        qk2 = jnp.concatenate([qk[sl], qk[sl]], axis=1)
        am = a * dmat + jnp.where(eye, qk2, f32(0.0))

        lhs = jnp.concatenate([am.astype(bf16), qs * qdec], axis=1)
        rhs = jnp.concatenate([vc, r], axis=0)
        out_ref[0, sl, :] = jnp.dot(lhs, rhs, preferred_element_type=f32)

        if c % half == half - 1 and c != N - 1:
            t = (c + 1) * C - 1
            qk4 = jnp.concatenate([qk[t:t + 1]] * 4, axis=1)
            out_ref[0, t:t + 1, :] = qk4 * v[t:t + 1].astype(f32)

        upd = jax.lax.dot_general(
            kc * kdec, vc, (((0,), (0,)), ((), ())),
            preferred_element_type=f32)
        r = upd.astype(bf16) + dc * r


@jax.jit
def kernel(xq, xk, xv, Wq, Wk, Wv):
    B, S, D = xq.shape
    C = _CHUNK
    G = B // _BPG
    S2 = S * _BPG
    xq = xq.reshape(G, S2, D)
    xk = xk.reshape(G, S2, D)
    xv = xv.reshape(G, S2, D)

    i = np.arange(C)
    dmat = np.where(i[:, None] >= i[None, :],
                    _DELTA ** (i[:, None] - i[None, :] + 1.0),
                    0.0).astype(np.float32)
    qdec = (_DELTA ** (i + 2.0)).astype(np.float32).reshape(C, 1)
    kdec = (_DELTA ** (C - 1.0 - i)).astype(np.float32).reshape(C, 1)

    def in_map(b):
        return (b, 0, 0)

    def w_map(b):
        return (0, 0)

    out = pl.pallas_call(
        _retention_body,
        grid=(G,),
        in_specs=[
            pl.BlockSpec((1, S2, D), in_map),
            pl.BlockSpec((1, S2, D), in_map),
            pl.BlockSpec((1, S2, D), in_map),
            pl.BlockSpec((D, D), w_map),
            pl.BlockSpec((D, D), w_map),
            pl.BlockSpec((D, D), w_map),
            pl.BlockSpec((C, C), w_map),
            pl.BlockSpec((C, 1), w_map),
            pl.BlockSpec((C, 1), w_map),
        ],
        out_specs=pl.BlockSpec((1, S2, D), in_map),
        out_shape=jax.ShapeDtypeStruct((G, S2, D), jnp.float32),
        scratch_shapes=[pltpu.VMEM((D, D + 128), jnp.bfloat16),
                        pltpu.VMEM((D, D + 128), jnp.bfloat16),
                        pltpu.VMEM((D, D), jnp.bfloat16)],
        compiler_params=pltpu.CompilerParams(
            dimension_semantics=("arbitrary",),
            vmem_limit_bytes=100 * 1024 * 1024,
        ),
        name="chunkwise_retention",
    )(xq, xk, xv, Wq, Wk, Wv,
      jnp.asarray(dmat), jnp.asarray(qdec), jnp.asarray(kdec))
    return out.reshape(B, S, D)


# speedup vs baseline: 1.0396x; 1.0396x over previous
import numpy as np
import jax
import jax.numpy as jnp
from jax.experimental import pallas as pl
from jax.experimental.pallas import tpu as pltpu

_GAMMA = 0.9865
_DELTA = _GAMMA * _GAMMA
_CHUNK = 256
_SEQ = 1024
_BPG = 1


def _retention_body(xq_ref, xk_ref, xv_ref, wq_ref, wk_ref, wv_ref,
                    dmat_ref, qdec_ref, kdec_ref, out_ref,
                    wqa_s, wka_s, wv_s):
    C = _CHUNK
    f32 = jnp.float32
    bf16 = jnp.bfloat16
    D = wq_ref.shape[0]

    @pl.when(pl.program_id(0) == 0)
    def _():
        ones = jnp.ones((D, 128), bf16)
        wq = wq_ref[...].astype(bf16)
        wk = wk_ref[...].astype(bf16)
        wqa_s[:, :D] = wq
        wqa_s[:, D:] = jnp.dot(wq, ones, preferred_element_type=f32
                               ).astype(bf16)
        wka_s[:, :D] = wk
        wka_s[:, D:] = jnp.dot(wk, ones, preferred_element_type=f32
                               ).astype(bf16)
        wv_s[...] = wv_ref[...].astype(bf16)

    qf = jnp.dot(xq_ref[0].astype(bf16), wqa_s[...],
                 preferred_element_type=f32)
    kf = jnp.dot(xk_ref[0].astype(bf16), wka_s[...],
                 preferred_element_type=f32)
    vf = jnp.dot(xv_ref[0].astype(bf16), wv_s[...],
                 preferred_element_type=f32)
    q = qf[:, :D].astype(bf16)
    k = kf[:, :D].astype(bf16)
    v = vf.astype(bf16)
    qk = qf[:, D:] * kf[:, D:]
    S = q.shape[0]
    N = S // C

    qs_all = jnp.concatenate([q[1:], jnp.zeros((1, D), bf16)], axis=0)

    rows = jax.lax.broadcasted_iota(jnp.int32, (C, C), 0)
    cols = jax.lax.broadcasted_iota(jnp.int32, (C, C), 1)
    eye = rows == cols
    dmat = dmat_ref[...].astype(bf16)
    qkb = qk.astype(bf16)
    qdec = qdec_ref[...].astype(bf16)
    kdec = kdec_ref[...].astype(bf16)
    dc = jnp.array(_DELTA ** C, bf16)
    half = _SEQ // C

    r = jnp.zeros((D, D), bf16)
    for c in range(N):
        if c % half == 0:
            r = jnp.zeros((D, D), bf16)
        sl = slice(c * C, (c + 1) * C)
        qs = qs_all[sl]
        kc = k[sl]
        vc = v[sl]

        a = jax.lax.dot_general(qs, kc, (((1,), (1,)), ((), ())),
                                preferred_element_type=f32)
        qk2 = jnp.concatenate([qkb[sl], qkb[sl]], axis=1)
        am = a.astype(bf16) * dmat + jnp.where(eye, qk2, jnp.array(0, bf16))

        lhs = jnp.concatenate([am, qs * qdec], axis=1)
        rhs = jnp.concatenate([vc, r], axis=0)
        out_ref[0, sl, :] = jnp.dot(lhs, rhs, preferred_element_type=f32)

        if c % half == half - 1 and c != N - 1:
            t = (c + 1) * C - 1
            qk4 = jnp.concatenate([qk[t:t + 1]] * 4, axis=1)
            out_ref[0, t:t + 1, :] = qk4 * v[t:t + 1].astype(f32)

        upd = jax.lax.dot_general(
            kc * kdec, vc, (((0,), (0,)), ((), ())),
            preferred_element_type=f32)
        r = upd.astype(bf16) + dc * r


@jax.jit
def kernel(xq, xk, xv, Wq, Wk, Wv):
    B, S, D = xq.shape
    C = _CHUNK
    G = B // _BPG
    S2 = S * _BPG
    xq = xq.reshape(G, S2, D)
    xk = xk.reshape(G, S2, D)
    xv = xv.reshape(G, S2, D)

    i = np.arange(C)
    dmat = np.where(i[:, None] >= i[None, :],
                    _DELTA ** (i[:, None] - i[None, :] + 1.0),
                    0.0).astype(np.float32)
    qdec = (_DELTA ** (i + 2.0)).astype(np.float32).reshape(C, 1)
    kdec = (_DELTA ** (C - 1.0 - i)).astype(np.float32).reshape(C, 1)

    def in_map(b):
        return (b, 0, 0)

    def w_map(b):
        return (0, 0)

    out = pl.pallas_call(
        _retention_body,
        grid=(G,),
        in_specs=[
            pl.BlockSpec((1, S2, D), in_map),
            pl.BlockSpec((1, S2, D), in_map),
            pl.BlockSpec((1, S2, D), in_map),
            pl.BlockSpec((D, D), w_map),
            pl.BlockSpec((D, D), w_map),
            pl.BlockSpec((D, D), w_map),
            pl.BlockSpec((C, C), w_map),
            pl.BlockSpec((C, 1), w_map),
            pl.BlockSpec((C, 1), w_map),
        ],
        out_specs=pl.BlockSpec((1, S2, D), in_map),
        out_shape=jax.ShapeDtypeStruct((G, S2, D), jnp.float32),
        scratch_shapes=[pltpu.VMEM((D, D + 128), jnp.bfloat16),
                        pltpu.VMEM((D, D + 128), jnp.bfloat16),
                        pltpu.VMEM((D, D), jnp.bfloat16)],
        compiler_params=pltpu.CompilerParams(
            dimension_semantics=("arbitrary",),
            vmem_limit_bytes=100 * 1024 * 1024,
        ),
        name="chunkwise_retention",
    )(xq, xk, xv, Wq, Wk, Wv,
      jnp.asarray(dmat), jnp.asarray(qdec), jnp.asarray(kdec))
    return out.reshape(B, S, D)


# speedup vs baseline: 1.0511x; 1.0110x over previous
import numpy as np
import jax
import jax.numpy as jnp
from jax.experimental import pallas as pl
from jax.experimental.pallas import tpu as pltpu

_GAMMA = 0.9865
_DELTA = _GAMMA * _GAMMA
_CHUNK = 256
_SEQ = 1024
_BPG = 1


def _retention_body(xq_ref, xk_ref, xv_ref, wq_ref, wk_ref, wv_ref,
                    dmat_ref, qdec_ref, kdec_ref, out_ref,
                    wqa_s, wka_s, wv_s):
    C = _CHUNK
    f32 = jnp.float32
    bf16 = jnp.bfloat16
    D = wq_ref.shape[0]

    @pl.when(pl.program_id(0) == 0)
    def _():
        ones = jnp.ones((D, 128), bf16)
        wq = wq_ref[...].astype(bf16)
        wk = wk_ref[...].astype(bf16)
        wqa_s[:, :D] = wq
        wqa_s[:, D:] = jnp.dot(wq, ones, preferred_element_type=f32
                               ).astype(bf16)
        wka_s[:, :D] = wk
        wka_s[:, D:] = jnp.dot(wk, ones, preferred_element_type=f32
                               ).astype(bf16)
        wv_s[...] = wv_ref[...].astype(bf16)

    qf = jnp.dot(xq_ref[0].astype(bf16), wqa_s[...],
                 preferred_element_type=f32)
    kf = jnp.dot(xk_ref[0].astype(bf16), wka_s[...],
                 preferred_element_type=f32)
    vf = jnp.dot(xv_ref[0].astype(bf16), wv_s[...],
                 preferred_element_type=f32)
    q = qf[:, :D].astype(bf16)
    k = kf[:, :D].astype(bf16)
    v = vf.astype(bf16)
    qk = qf[:, D:] * kf[:, D:]
    S = q.shape[0]
    N = S // C

    qs_all = jnp.concatenate([q[1:], jnp.zeros((1, D), bf16)], axis=0)

    rows = jax.lax.broadcasted_iota(jnp.int32, (C, C), 0)
    cols = jax.lax.broadcasted_iota(jnp.int32, (C, C), 1)
    eye = rows == cols
    dmat = dmat_ref[...]
    qdec = qdec_ref[...].astype(bf16)
    kdec = kdec_ref[...].astype(bf16)
    dc = jnp.array(_DELTA ** C, bf16)
    half = _SEQ // C

    r = jnp.zeros((D, D), bf16)
    for c in range(N):
        if c % half == 0:
            r = jnp.zeros((D, D), bf16)
        sl = slice(c * C, (c + 1) * C)
        qs = qs_all[sl]
        kc = k[sl]
        vc = v[sl]

        a = jax.lax.dot_general(qs, kc, (((1,), (1,)), ((), ())),
                                preferred_element_type=f32)
        qk2 = jnp.concatenate([qk[sl], qk[sl]], axis=1)
        am = a * dmat + jnp.where(eye, qk2, f32(0.0))

        lhs = jnp.concatenate([am.astype(bf16), qs * qdec], axis=1)
        rhs = jnp.concatenate([vc, r], axis=0)
        out_ref[0, sl, :] = jnp.dot(lhs, rhs, preferred_element_type=f32)

        if c % half == half - 1 and c != N - 1:
            t = (c + 1) * C - 1
            qk4 = jnp.concatenate([qk[t:t + 1]] * 4, axis=1)
            out_ref[0, t:t + 1, :] = qk4 * v[t:t + 1].astype(f32)

        upd = jax.lax.dot_general(
            kc * kdec, vc, (((0,), (0,)), ((), ())),
            preferred_element_type=f32)
        r = upd.astype(bf16) + dc * r


@jax.jit
def kernel(xq, xk, xv, Wq, Wk, Wv):
    B, S, D = xq.shape
    C = _CHUNK
    G = B // _BPG
    S2 = S * _BPG
    xq = xq.reshape(G, S2, D)
    xk = xk.reshape(G, S2, D)
    xv = xv.reshape(G, S2, D)

    i = np.arange(C)
    dmat = np.where(i[:, None] >= i[None, :],
                    _DELTA ** (i[:, None] - i[None, :] + 1.0),
                    0.0).astype(np.float32)
    qdec = (_DELTA ** (i + 2.0)).astype(np.float32).reshape(C, 1)
    kdec = (_DELTA ** (C - 1.0 - i)).astype(np.float32).reshape(C, 1)

    def in_map(b):
        return (b, 0, 0)

    def w_map(b):
        return (0, 0)

    out = pl.pallas_call(
        _retention_body,
        grid=(G,),
        in_specs=[
            pl.BlockSpec((1, S2, D), in_map),
            pl.BlockSpec((1, S2, D), in_map),
            pl.BlockSpec((1, S2, D), in_map),
            pl.BlockSpec((D, D), w_map),
            pl.BlockSpec((D, D), w_map),
            pl.BlockSpec((D, D), w_map),
            pl.BlockSpec((C, C), w_map),
            pl.BlockSpec((C, 1), w_map),
            pl.BlockSpec((C, 1), w_map),
        ],
        out_specs=pl.BlockSpec((1, S2, D), in_map),
        out_shape=jax.ShapeDtypeStruct((G, S2, D), jnp.float32),
        scratch_shapes=[pltpu.VMEM((D, D + 128), jnp.bfloat16),
                        pltpu.VMEM((D, D + 128), jnp.bfloat16),
                        pltpu.VMEM((D, D), jnp.bfloat16)],
        compiler_params=pltpu.CompilerParams(
            dimension_semantics=("arbitrary",),
            vmem_limit_bytes=100 * 1024 * 1024,
        ),
        name="chunkwise_retention",
    )(xq, xk, xv, Wq, Wk, Wv,
      jnp.asarray(dmat), jnp.asarray(qdec), jnp.asarray(kdec))
    return out.reshape(B, S, D)
